# Initial kernel scaffold; baseline (speedup 1.0000x reference)
import jax
import jax.numpy as jnp
from jax.experimental import pallas as pl


def kernel(x, x_wave, encoder_padding):
    raise NotImplementedError("write your pallas kernel here")


# speedup vs baseline: 2.1394x; 2.1394x over previous
import jax
import jax.numpy as jnp
from jax import lax
from jax.experimental import pallas as pl
from jax.experimental.pallas import tpu as pltpu
from jax.experimental.pallas import tpu_sc as plsc

B, C, N, L = 8, 2, 256, 4000
M = 128
MCHUNK = 32
F = 200
NT = L // F
OUT_LEN = M * (L + 1) - 1


def _sc_body(x_hbm, xw_hbm, out_hbm,
             xl0, xu0, xl1, xu1, wl, wu, ob0, ob1, tail):
    cid = lax.axis_index("c")
    sid = lax.axis_index("s")
    wid = sid * 2 + cid
    b = wid // 4
    m0 = (wid % 4) * MCHUNK
    iota = lax.iota(jnp.int32, 16)
    rows_h = (iota, iota + 16)
    zero = jnp.zeros((16,), jnp.float32)

    xbufs = ((xl0, xu0), (xl1, xu1))
    obufs = (ob0, ob1)

    pu = (zero, zero, zero, zero)

    for t in range(NT):
        k0 = t * F
        pltpu.sync_copy(x_hbm.at[b, 0, pl.ds(m0, MCHUNK), pl.ds(k0, F)], xl0)
        pltpu.sync_copy(x_hbm.at[b, 0, pl.ds(128 + m0, MCHUNK), pl.ds(k0, F)], xu0)
        pltpu.sync_copy(x_hbm.at[b, 1, pl.ds(m0, MCHUNK), pl.ds(k0, F)], xl1)
        pltpu.sync_copy(x_hbm.at[b, 1, pl.ds(128 + m0, MCHUNK), pl.ds(k0, F)], xu1)
        pltpu.sync_copy(xw_hbm.at[b, pl.ds(m0, MCHUNK), pl.ds(k0, F)], wl)
        pltpu.sync_copy(xw_hbm.at[b, pl.ds(128 + m0, MCHUNK), pl.ds(k0, F)], wu)

        def body(j, carry, _xbufs=xbufs, _obufs=obufs):
            col = jnp.full((16,), j, jnp.int32)
            new = list(carry)
            for h in range(2):
                rows = rows_h[h]
                wlv = plsc.load_gather(wl, [rows, col])
                wuv = plsc.load_gather(wu, [rows, col])
                for c in range(2):
                    xlb, xub = _xbufs[c]
                    xlv = plsc.load_gather(xlb, [rows, col])
                    xuv = plsc.load_gather(xub, [rows, col])
                    _obufs[c][j, pl.ds(h * 16, 16)] = xlv * wlv + carry[2 * c + h]
                    new[2 * c + h] = xuv * wuv
            return tuple(new)

        pu = lax.fori_loop(0, F, body, pu)

        pltpu.sync_copy(ob0, out_hbm.at[b, 0, pl.ds(k0, F), pl.ds(m0, MCHUNK)])
        pltpu.sync_copy(ob1, out_hbm.at[b, 1, pl.ds(k0, F), pl.ds(m0, MCHUNK)])

    for c in range(2):
        tail[0, pl.ds(0, 16)] = pu[2 * c + 0]
        tail[0, pl.ds(16, 16)] = pu[2 * c + 1]
        pltpu.sync_copy(tail, out_hbm.at[b, c, pl.ds(L, 1), pl.ds(m0, MCHUNK)])


import functools


@functools.lru_cache(maxsize=1)
def _oadd():
    return pl.kernel(
        _sc_body,
        out_type=jax.ShapeDtypeStruct((B, C, L + 1, M), jnp.float32),
        mesh=plsc.VectorSubcoreMesh(core_axis_name="c", subcore_axis_name="s"),
        scratch_types=(
            [pltpu.VMEM((MCHUNK, F), jnp.float32)] * 6
            + [pltpu.VMEM((F, MCHUNK), jnp.float32)] * 2
            + [pltpu.VMEM((1, MCHUNK), jnp.float32)]
        ),
        compiler_params=pltpu.CompilerParams(use_tc_tiling_on_sc=False,
                                             needs_layout_passes=False),
    )


def kernel(x, x_wave, encoder_padding):
    out4 = _oadd()(x, x_wave)
    y = out4.reshape(B, C, (L + 1) * M)
    ep = encoder_padding.astype(jnp.int32)
    start = ep[0] + ep[1] - 1
    return lax.dynamic_slice_in_dim(y, start, OUT_LEN, axis=2)
